# baseline (speedup 1.0000x reference)
import functools

import jax
import jax.numpy as jnp
from jax import lax
from jax.experimental import pallas as pl
from jax.experimental.pallas import tpu as pltpu
from jax.experimental.pallas import tpu_sc as plsc

_NC, _NS = 2, 16
_NW = _NC * _NS

_NJ = 200
_TB = 128
_D = 64

_mesh = plsc.VectorSubcoreMesh(core_axis_name="c", subcore_axis_name="s")


@functools.partial(
    pl.kernel,
    out_type=jax.ShapeDtypeStruct((_NJ, 8, _NW, 8 * _TB), jnp.float32),
    mesh=_mesh,
    scratch_types=[
        pltpu.VMEM((_NJ, _TB), jnp.int32),
        pltpu.VMEM((2, _TB, _D), jnp.float32),
        pltpu.VMEM((2, 8, 8 * _TB), jnp.float32),
        pltpu.SemaphoreType.DMA,
        pltpu.SemaphoreType.DMA,
        pltpu.SemaphoreType.DMA,
        pltpu.SemaphoreType.DMA,
    ],
    compiler_params=pltpu.CompilerParams(use_tc_tiling_on_sc=False,
                                         needs_layout_passes=False),
)
def _gather_kernel(tok_hbm, table_hbm, out_hbm, idx_all, rows, tbuf,
                   g0, g1, s0, s1):
    gsem = (g0, g1)
    ssem = (s0, s1)
    wid = lax.axis_index("s") * _NC + lax.axis_index("c")
    pltpu.sync_copy(tok_hbm.at[:, wid], idx_all)

    iota = lax.iota(jnp.int32, 16)
    k8 = [jnp.right_shift(k0 * 16 + iota, 3) for k0 in range(4)]
    kin = [jnp.left_shift(jnp.bitwise_and(k0 * 16 + iota, 7), 7)
           for k0 in range(4)]

    def start_gather(j, b):
        pltpu.async_copy(table_hbm.at[idx_all.at[j]], rows.at[b], gsem[b])

    def wait_gather(j, b):
        pltpu.make_async_copy(table_hbm.at[idx_all.at[j]], rows.at[b],
                              gsem[b]).wait()

    def start_store(j, b):
        pltpu.async_copy(tbuf.at[b], out_hbm.at[j, :, wid], ssem[b])

    def wait_store(j, b):
        pltpu.make_async_copy(tbuf.at[b], out_hbm.at[j, :, wid],
                              ssem[b]).wait()

    def transpose(b):
        for l in range(_TB):
            for k0 in range(4):
                v = rows[b, l, pl.ds(k0 * 16, 16)]
                plsc.store_scatter(tbuf.at[b], [k8[k0], kin[k0] + l], v)

    start_gather(0, 0)

    def pair_body(p, carry):
        for b in (0, 1):
            j = 2 * p + b
            wait_gather(j, b)

            @pl.when(j + 1 < _NJ)
            def _():
                start_gather(j + 1, 1 - b)

            @pl.when(j >= 2)
            def _():
                wait_store(j - 2, b)

            transpose(b)
            start_store(j, b)
        return carry

    lax.fori_loop(0, _NJ // 2, pair_body, 0)
    wait_store(_NJ - 2, 0)
    wait_store(_NJ - 1, 1)


def kernel(tokens, embedding_weight):
    tt = tokens.T.reshape(_NJ, _NW, _TB).astype(jnp.int32)
    buf = _gather_kernel(tt, embedding_weight)
    buf = buf.reshape(_NJ, 8, _NW, 8, _TB)
    return buf.transpose(2, 4, 0, 1, 3).reshape(_NW * _TB, _NJ, _D)


# speedup vs baseline: 1.0063x; 1.0063x over previous
import functools

import jax
import jax.numpy as jnp
from jax import lax
from jax.experimental import pallas as pl
from jax.experimental.pallas import tpu as pltpu
from jax.experimental.pallas import tpu_sc as plsc

_NC, _NS = 2, 16
_NW = _NC * _NS

_NJ = 200
_TB = 128
_D = 64
_NG = 4

_mesh = plsc.VectorSubcoreMesh(core_axis_name="c", subcore_axis_name="s")


@functools.partial(
    pl.kernel,
    out_type=jax.ShapeDtypeStruct((_NJ, 8, _NW, 8 * _TB), jnp.float32),
    mesh=_mesh,
    scratch_types=[
        pltpu.VMEM((_NJ, _TB), jnp.int32),
        pltpu.VMEM((_NG, _TB, _D), jnp.float32),
        pltpu.VMEM((2, 8, 8 * _TB), jnp.float32),
        pltpu.SemaphoreType.DMA,
        pltpu.SemaphoreType.DMA,
        pltpu.SemaphoreType.DMA,
        pltpu.SemaphoreType.DMA,
        pltpu.SemaphoreType.DMA,
        pltpu.SemaphoreType.DMA,
    ],
    compiler_params=pltpu.CompilerParams(use_tc_tiling_on_sc=False,
                                         needs_layout_passes=False),
)
def _gather_kernel(tok_hbm, table_hbm, out_hbm, idx_all, rows, tbuf,
                   g0, g1, g2, g3, s0, s1):
    gsem = (g0, g1, g2, g3)
    ssem = (s0, s1)
    wid = lax.axis_index("s") * _NC + lax.axis_index("c")
    pltpu.sync_copy(tok_hbm.at[:, wid], idx_all)

    iota = lax.iota(jnp.int32, 16)
    k8 = [jnp.right_shift(k0 * 16 + iota, 3) for k0 in range(4)]
    kin = [jnp.left_shift(jnp.bitwise_and(k0 * 16 + iota, 7), 7)
           for k0 in range(4)]

    def start_gather(j, b):
        pltpu.async_copy(table_hbm.at[idx_all.at[j]], rows.at[b], gsem[b])

    def wait_gather(j, b):
        pltpu.make_async_copy(table_hbm.at[idx_all.at[j]], rows.at[b],
                              gsem[b]).wait()

    def start_store(j, b):
        pltpu.async_copy(tbuf.at[b], out_hbm.at[j, :, wid], ssem[b])

    def wait_store(j, b):
        pltpu.make_async_copy(tbuf.at[b], out_hbm.at[j, :, wid],
                              ssem[b]).wait()

    def transpose(rb, tb):
        def tr_body(c, carry):
            for li in range(16):
                l = c * 16 + li
                for k0 in range(4):
                    v = rows[rb, l, pl.ds(k0 * 16, 16)]
                    plsc.store_scatter(tbuf.at[tb], [k8[k0], kin[k0] + l], v)
            return carry
        lax.fori_loop(0, _TB // 16, tr_body, 0)

    for j in range(_NG):
        start_gather(j, j)

    def body(q, carry):
        for u in range(_NG):
            j = _NG * q + u
            tb = u % 2
            wait_gather(j, u)

            @pl.when(j >= 2)
            def _():
                wait_store(j - 2, tb)

            transpose(u, tb)

            @pl.when(j + _NG < _NJ)
            def _():
                start_gather(j + _NG, u)

            start_store(j, tb)
        return carry

    lax.fori_loop(0, _NJ // _NG, body, 0)
    wait_store(_NJ - 2, 0)
    wait_store(_NJ - 1, 1)


def kernel(tokens, embedding_weight):
    tt = tokens.T.reshape(_NJ, _NW, _TB).astype(jnp.int32)
    buf = _gather_kernel(tt, embedding_weight)
    buf = buf.reshape(_NJ, 8, _NW, 8, _TB)
    return buf.transpose(2, 4, 0, 1, 3).reshape(_NW * _TB, _NJ, _D)


# speedup vs baseline: 1.5843x; 1.5743x over previous
import functools

import jax
import jax.numpy as jnp
from jax import lax
from jax.experimental import pallas as pl
from jax.experimental.pallas import tpu as pltpu
from jax.experimental.pallas import tpu_sc as plsc

_NC, _NS = 2, 16
_NW = _NC * _NS

_NJ = 200
_TB = 128
_D = 64
_NG = 4

_mesh = plsc.VectorSubcoreMesh(core_axis_name="c", subcore_axis_name="s")


@functools.partial(
    pl.kernel,
    out_type=jax.ShapeDtypeStruct((_NJ, 8, _NW, 8, _TB), jnp.float32),
    mesh=_mesh,
    scratch_types=[
        pltpu.VMEM((_NJ, _TB), jnp.int32),
        pltpu.VMEM((_NG, _TB, _D), jnp.float32),
        pltpu.VMEM((2, 8, 8, _TB + 1), jnp.float32),
        pltpu.SemaphoreType.DMA,
        pltpu.SemaphoreType.DMA,
        pltpu.SemaphoreType.DMA,
        pltpu.SemaphoreType.DMA,
        pltpu.SemaphoreType.DMA,
        pltpu.SemaphoreType.DMA,
    ],
    compiler_params=pltpu.CompilerParams(use_tc_tiling_on_sc=False,
                                         needs_layout_passes=False),
)
def _gather_kernel(tok_hbm, table_hbm, out_hbm, idx_all, rows, tbuf,
                   g0, g1, g2, g3, s0, s1):
    gsem = (g0, g1, g2, g3)
    ssem = (s0, s1)
    wid = lax.axis_index("s") * _NC + lax.axis_index("c")
    pltpu.sync_copy(tok_hbm.at[:, wid], idx_all)

    iota = lax.iota(jnp.int32, 16)
    k8 = [jnp.right_shift(k0 * 16 + iota, 3) for k0 in range(4)]
    k3 = [jnp.bitwise_and(k0 * 16 + iota, 7) for k0 in range(4)]

    def start_gather(j, b):
        pltpu.async_copy(table_hbm.at[idx_all.at[j]], rows.at[b], gsem[b])

    def wait_gather(j, b):
        pltpu.make_async_copy(table_hbm.at[idx_all.at[j]], rows.at[b],
                              gsem[b]).wait()

    def start_store(j, b):
        pltpu.async_copy(tbuf.at[b, :, :, pl.ds(0, _TB)],
                         out_hbm.at[j, :, wid], ssem[b])

    def wait_store(j, b):
        pltpu.make_async_copy(tbuf.at[b, :, :, pl.ds(0, _TB)],
                              out_hbm.at[j, :, wid], ssem[b]).wait()

    def transpose(rb, tb):
        def tr_body(c, carry):
            for li in range(16):
                l = c * 16 + li
                lv = jnp.zeros((16,), jnp.int32) + l
                for k0 in range(4):
                    v = rows[rb, l, pl.ds(k0 * 16, 16)]
                    plsc.store_scatter(tbuf.at[tb], [k8[k0], k3[k0], lv], v)
            return carry
        lax.fori_loop(0, _TB // 16, tr_body, 0)

    for j in range(_NG):
        start_gather(j, j)

    def body(q, carry):
        for u in range(_NG):
            j = _NG * q + u
            tb = u % 2
            wait_gather(j, u)

            @pl.when(j >= 2)
            def _():
                wait_store(j - 2, tb)

            transpose(u, tb)

            @pl.when(j + _NG < _NJ)
            def _():
                start_gather(j + _NG, u)

            start_store(j, tb)
        return carry

    lax.fori_loop(0, _NJ // _NG, body, 0)
    wait_store(_NJ - 2, 0)
    wait_store(_NJ - 1, 1)


def kernel(tokens, embedding_weight):
    tt = tokens.T.reshape(_NJ, _NW, _TB).astype(jnp.int32)
    buf = _gather_kernel(tt, embedding_weight)
    return buf.transpose(2, 4, 0, 1, 3).reshape(_NW * _TB, _NJ, _D)


# speedup vs baseline: 1.5903x; 1.0038x over previous
import functools

import jax
import jax.numpy as jnp
from jax import lax
from jax.experimental import pallas as pl
from jax.experimental.pallas import tpu as pltpu
from jax.experimental.pallas import tpu_sc as plsc

_NC, _NS = 2, 16
_NW = _NC * _NS

_NJ = 200
_TB = 128
_D = 64
_NG = 4

_mesh = plsc.VectorSubcoreMesh(core_axis_name="c", subcore_axis_name="s")


@functools.partial(
    pl.kernel,
    out_type=jax.ShapeDtypeStruct((_NJ, 8, _NW, 8, _TB), jnp.float32),
    mesh=_mesh,
    scratch_types=[
        pltpu.VMEM((_NJ // 8, 8, _TB), jnp.int32),
        pltpu.VMEM((_NG, _TB, _D), jnp.float32),
        pltpu.VMEM((2, 8, 8, _TB + 1), jnp.float32),
        pltpu.SemaphoreType.DMA,
        pltpu.SemaphoreType.DMA,
        pltpu.SemaphoreType.DMA,
        pltpu.SemaphoreType.DMA,
        pltpu.SemaphoreType.DMA,
        pltpu.SemaphoreType.DMA,
    ],
    compiler_params=pltpu.CompilerParams(use_tc_tiling_on_sc=False,
                                         needs_layout_passes=False),
)
def _gather_kernel(tok_hbm, table_hbm, out_hbm, idx_all, rows, tbuf,
                   g0, g1, g2, g3, s0, s1):
    gsem = (g0, g1, g2, g3)
    ssem = (s0, s1)
    wid = lax.axis_index("s") * _NC + lax.axis_index("c")
    pltpu.sync_copy(tok_hbm.at[:, wid, :, :], idx_all)

    iota = lax.iota(jnp.int32, 16)
    k8 = [jnp.right_shift(k0 * 16 + iota, 3) for k0 in range(4)]
    k3 = [jnp.bitwise_and(k0 * 16 + iota, 7) for k0 in range(4)]

    def start_gather(j, b):
        pltpu.async_copy(table_hbm.at[idx_all.at[j // 8, j % 8]], rows.at[b],
                         gsem[b])

    def wait_gather(j, b):
        pltpu.make_async_copy(table_hbm.at[idx_all.at[j // 8, j % 8]],
                              rows.at[b], gsem[b]).wait()

    def start_store(j, b):
        pltpu.async_copy(tbuf.at[b, :, :, pl.ds(0, _TB)],
                         out_hbm.at[j, :, wid], ssem[b])

    def wait_store(j, b):
        pltpu.make_async_copy(tbuf.at[b, :, :, pl.ds(0, _TB)],
                              out_hbm.at[j, :, wid], ssem[b]).wait()

    def transpose(rb, tb):
        def tr_body(c, carry):
            for li in range(16):
                l = c * 16 + li
                lv = jnp.zeros((16,), jnp.int32) + l
                for k0 in range(4):
                    v = rows[rb, l, pl.ds(k0 * 16, 16)]
                    plsc.store_scatter(tbuf.at[tb], [k8[k0], k3[k0], lv], v)
            return carry
        lax.fori_loop(0, _TB // 16, tr_body, 0)

    for j in range(_NG):
        start_gather(j, j)

    def body(q, carry):
        for u in range(_NG):
            j = _NG * q + u
            tb = u % 2
            wait_gather(j, u)

            @pl.when(j >= 2)
            def _():
                wait_store(j - 2, tb)

            transpose(u, tb)

            @pl.when(j + _NG < _NJ)
            def _():
                start_gather(j + _NG, u)

            start_store(j, tb)
        return carry

    lax.fori_loop(0, _NJ // _NG, body, 0)
    wait_store(_NJ - 2, 0)
    wait_store(_NJ - 1, 1)


def kernel(tokens, embedding_weight):
    tt = (tokens.T.reshape(_NJ // 8, 8, _NW, _TB)
          .transpose(0, 2, 1, 3).astype(jnp.int32))
    buf = _gather_kernel(tt, embedding_weight)
    return buf.transpose(2, 4, 0, 1, 3).reshape(_NW * _TB, _NJ, _D)
